# Initial kernel scaffold; baseline (speedup 1.0000x reference)
import jax
import jax.numpy as jnp
from jax.experimental import pallas as pl


def kernel(x, edge_index, W0, W1):
    raise NotImplementedError("write your pallas kernel here")


# speedup vs baseline: 2.5132x; 2.5132x over previous
import functools

import jax
import jax.numpy as jnp
from jax import lax
from jax.experimental import pallas as pl
from jax.experimental.pallas import tpu as pltpu
from jax.experimental.pallas import tpu_sc as plsc

N = 10000
NP = 10240
E = 320000
EP = 327680
F = 128
N_CLS = 64

NC = 2
NS = 16
NW = NC * NS
EPW = EP // NW
CHUNK = 128
NCHUNK = EPW // CHUNK
IB = 16
ROWS_PT = NP // NS

_sc_mesh = plsc.VectorSubcoreMesh(core_axis_name="c", subcore_axis_name="s")


@functools.partial(
    pl.kernel,
    out_type=jax.ShapeDtypeStruct((NC, NP, F), jnp.float32),
    mesh=_sc_mesh,
    scratch_types=[
        pltpu.VMEM((NCHUNK, CHUNK), jnp.int32),
        pltpu.VMEM((IB, CHUNK), jnp.int32),
        pltpu.VMEM((CHUNK, F), jnp.float32),
        pltpu.VMEM((CHUNK, F), jnp.float32),
        pltpu.VMEM_SHARED((NP, F), jnp.float32),
        pltpu.SemaphoreType.DMA,
        pltpu.SemaphoreType.DMA,
    ],
)
def _hop_kernel(g_hbm, src_hbm, dst_hbm, zeros_hbm, out_hbm,
                src_v, dst_v, rows0, rows1, acc, sem0, sem1):
    cid = lax.axis_index("c")
    sid = lax.axis_index("s")
    wid = sid * NC + cid
    r0 = sid * ROWS_PT
    pltpu.sync_copy(zeros_hbm.at[pl.ds(r0, ROWS_PT)], acc.at[pl.ds(r0, ROWS_PT)])
    pltpu.sync_copy(src_hbm.at[wid], src_v)
    pltpu.sync_copy(dst_hbm.at[wid, pl.ds(0, IB)], dst_v)
    plsc.subcore_barrier()

    pltpu.async_copy(g_hbm.at[src_v.at[0]], rows0, sem0)

    def step(k, rows_cur, sem_cur, rows_nxt, sem_nxt):
        pltpu.make_async_copy(g_hbm.at[src_v.at[k]], rows_cur, sem_cur).wait()

        @pl.when(k + 1 < NCHUNK)
        def _():
            pltpu.async_copy(g_hbm.at[src_v.at[k + 1]], rows_nxt, sem_nxt)
        pltpu.sync_copy(rows_cur, acc.at[dst_v.at[k % IB]], add=True)

    def body(k, carry):
        @pl.when(jnp.logical_and(k % IB == 0, k > 0))
        def _():
            pltpu.sync_copy(dst_hbm.at[wid, pl.ds(pl.multiple_of((k // IB) * IB, IB), IB)],
                            dst_v)

        @pl.when(k % 2 == 0)
        def _():
            step(k, rows0, sem0, rows1, sem1)

        @pl.when(k % 2 == 1)
        def _():
            step(k, rows1, sem1, rows0, sem0)
        return carry

    lax.fori_loop(0, NCHUNK, body, 0)
    plsc.subcore_barrier()
    pltpu.sync_copy(acc.at[pl.ds(r0, ROWS_PT)], out_hbm.at[cid, pl.ds(r0, ROWS_PT)])


_BN = 2048
_GRID = NP // _BN


def _prep_body(degp_ref, x_ref, norm_ref, g_ref):
    deg = degp_ref[0, :, 0:1] + degp_ref[1, :, 0:1]
    nrm = lax.rsqrt(jnp.maximum(deg, 1.0))
    nb = jnp.broadcast_to(nrm, x_ref.shape)
    norm_ref[...] = nb
    g_ref[...] = x_ref[...] * nb


_prep = pl.pallas_call(
    _prep_body,
    grid=(_GRID,),
    in_specs=[
        pl.BlockSpec((NC, _BN, F), lambda i: (0, i, 0)),
        pl.BlockSpec((_BN, F), lambda i: (i, 0)),
    ],
    out_specs=[
        pl.BlockSpec((_BN, F), lambda i: (i, 0)),
        pl.BlockSpec((_BN, F), lambda i: (i, 0)),
    ],
    out_shape=[
        jax.ShapeDtypeStruct((NP, F), jnp.float32),
        jax.ShapeDtypeStruct((NP, F), jnp.float32),
    ],
)


def _mid_body(p_ref, n_ref, g_ref):
    n = n_ref[...]
    g_ref[...] = (p_ref[0] + p_ref[1]) * n * n


_mid = pl.pallas_call(
    _mid_body,
    grid=(_GRID,),
    in_specs=[
        pl.BlockSpec((NC, _BN, F), lambda i: (0, i, 0)),
        pl.BlockSpec((_BN, F), lambda i: (i, 0)),
    ],
    out_specs=pl.BlockSpec((_BN, F), lambda i: (i, 0)),
    out_shape=jax.ShapeDtypeStruct((NP, F), jnp.float32),
)


def _mm0_body(p_ref, n_ref, w_ref, o_ref):
    n = n_ref[...]
    h = (p_ref[0] + p_ref[1]) * n
    h = jnp.dot(h, w_ref[...], preferred_element_type=jnp.float32)
    o_ref[...] = jnp.maximum(h, 0.0) * n


_mm0 = pl.pallas_call(
    _mm0_body,
    grid=(_GRID,),
    in_specs=[
        pl.BlockSpec((NC, _BN, F), lambda i: (0, i, 0)),
        pl.BlockSpec((_BN, F), lambda i: (i, 0)),
        pl.BlockSpec((F, F), lambda i: (0, 0)),
    ],
    out_specs=pl.BlockSpec((_BN, F), lambda i: (i, 0)),
    out_shape=jax.ShapeDtypeStruct((NP, F), jnp.float32),
)


def _mm1_body(p_ref, n_ref, w_ref, o_ref):
    h = (p_ref[0] + p_ref[1]) * n_ref[...]
    o_ref[...] = jnp.dot(h, w_ref[...], preferred_element_type=jnp.float32)


_mm1 = pl.pallas_call(
    _mm1_body,
    grid=(_GRID,),
    in_specs=[
        pl.BlockSpec((NC, _BN, F), lambda i: (0, i, 0)),
        pl.BlockSpec((_BN, F), lambda i: (i, 0)),
        pl.BlockSpec((F, N_CLS), lambda i: (0, 0)),
    ],
    out_specs=pl.BlockSpec((_BN, N_CLS), lambda i: (i, 0)),
    out_shape=jax.ShapeDtypeStruct((NP, N_CLS), jnp.float32),
)


def kernel(x, edge_index, W0, W1):
    pad = EP - E
    src = jnp.concatenate([edge_index[0], jnp.zeros((pad,), jnp.int32)])
    dst = jnp.concatenate([edge_index[1], jnp.full((pad,), N, jnp.int32)])
    src = src.reshape(NW, NCHUNK, CHUNK)
    dst = dst.reshape(NW, NCHUNK, CHUNK)
    zeros_f = jnp.zeros((NP, F), jnp.float32)
    ones_f = jnp.ones((NP, F), jnp.float32)

    x_p = jnp.concatenate([x, jnp.zeros((NP - N, F), jnp.float32)])
    degp = _hop_kernel(ones_f, src, dst, zeros_f)
    norm_b, g = _prep(degp, x_p)

    p = _hop_kernel(g, src, dst, zeros_f)
    g = _mid(p, norm_b)
    p = _hop_kernel(g, src, dst, zeros_f)
    g = _mm0(p, norm_b, W0)

    p = _hop_kernel(g, src, dst, zeros_f)
    g = _mid(p, norm_b)
    p = _hop_kernel(g, src, dst, zeros_f)
    return _mm1(p, norm_b, W1)[:N]
